# baseline (speedup 1.0000x reference)
import jax
import jax.numpy as jnp
from jax.experimental import pallas as pl
from jax.experimental.pallas import tpu as pltpu

N_IN = 16384
N_OUT = 128
B = 1024
BBLK = 128
NBB = B // BBLK


def _spw_kernel(x_ref, w_ref, idx_ref, gamma_ref, beta_ref, co_ref,
                out_ref, acc_ref, oh_ref):
    k = pl.program_id(0)

    @pl.when(k == 0)
    def _build_onehot():
        idxv = idx_ref[0, 0, :]
        oh_ref[...] = jnp.where(
            idxv[:, None] == jax.lax.broadcasted_iota(jnp.int32, (N_IN, N_OUT), 1),
            w_ref[0, :][:, None],
            0.0,
        )

    rows = pl.ds(k * BBLK, BBLK)
    acc_ref[rows, :] = jnp.dot(
        x_ref[...], oh_ref[...], preferred_element_type=jnp.float32
    )

    @pl.when(k == NBB - 1)
    def _finish():
        h = jnp.maximum(acc_ref[...], 0.0)
        mean = jnp.mean(h, axis=0, keepdims=True)
        d = h - mean
        var = jnp.mean(d * d, axis=0, keepdims=True)
        hn = d * jax.lax.rsqrt(var + 1e-5) * gamma_ref[...] + beta_ref[...]
        out_ref[...] = hn * jax.nn.sigmoid(co_ref[...])


@jax.jit
def kernel(x, weight, gamma, beta, co_weight, idx):
    idx3 = idx.astype(jnp.int32).reshape(1, 1, N_IN)
    gamma2 = gamma.reshape(1, N_OUT)
    beta2 = beta.reshape(1, N_OUT)
    co2 = co_weight.reshape(1, N_OUT)
    return pl.pallas_call(
        _spw_kernel,
        grid=(NBB,),
        in_specs=[
            pl.BlockSpec((BBLK, N_IN), lambda k: (k, 0)),
            pl.BlockSpec((1, N_IN), lambda k: (0, 0)),
            pl.BlockSpec((1, 1, N_IN), lambda k: (0, 0, 0)),
            pl.BlockSpec((1, N_OUT), lambda k: (0, 0)),
            pl.BlockSpec((1, N_OUT), lambda k: (0, 0)),
            pl.BlockSpec((1, N_OUT), lambda k: (0, 0)),
        ],
        out_specs=pl.BlockSpec((B, N_OUT), lambda k: (0, 0)),
        out_shape=jax.ShapeDtypeStruct((B, N_OUT), jnp.float32),
        scratch_shapes=[
            pltpu.VMEM((B, N_OUT), jnp.float32),
            pltpu.VMEM((N_IN, N_OUT), jnp.float32),
        ],
    )(x, weight, idx3, gamma2, beta2, co2)
